# baseline (speedup 1.0000x reference)
import jax
import jax.numpy as jnp
from jax import lax
from jax.experimental import pallas as pl
from jax.experimental.pallas import tpu as pltpu

HIDDEN = 64
MAX_DEPTH = 8
NUM_TYPES = 16
K = 32

BLK_TOK = 25600
N_CHUNKS = 4


def _compute(comb_ref, dtab_ref, ttab_ref, w_ref, b_ref, out_ref):
    w = w_ref[...]
    pd = lax.dot_general(dtab_ref[...], w[:, :HIDDEN],
                         (((1,), (1,)), ((), ())),
                         preferred_element_type=jnp.float32) + b_ref[...]
    pt = lax.dot_general(ttab_ref[...], w[:, HIDDEN:],
                         (((1,), (1,)), ((), ())),
                         preferred_element_type=jnp.float32)
    ptab = jnp.concatenate(
        [pd, pt, jnp.zeros((K - MAX_DEPTH - NUM_TYPES, HIDDEN), jnp.float32)], axis=0)

    row = comb_ref[0]
    d = jnp.broadcast_to(row & (MAX_DEPTH - 1), (K, BLK_TOK))
    c = jnp.broadcast_to((row >> 3) + MAX_DEPTH, (K, BLK_TOK))
    iota = lax.broadcasted_iota(jnp.int32, (K, BLK_TOK), 0)
    two_hot_t = jnp.where((iota == d) | (iota == c), 1.0, 0.0)
    out_ref[...] = lax.dot_general(two_hot_t, ptab,
                                   (((0,), (0,)), ((), ())),
                                   preferred_element_type=jnp.float32)


def _body_first(comb_ref, dtab_ref, ttab_ref, w_ref, b_ref, out_ref):
    _compute(comb_ref, dtab_ref, ttab_ref, w_ref, b_ref, out_ref)


def _body_chained(comb_ref, dtab_ref, ttab_ref, w_ref, b_ref, prev_ref, out_ref):
    del prev_ref
    _compute(comb_ref, dtab_ref, ttab_ref, w_ref, b_ref, out_ref)


def kernel(depth_indices, node_type_indices, depth_table, type_table, W, b):
    B, L = depth_indices.shape
    n_tok = B * L
    grid = n_tok // BLK_TOK
    grid_c = grid // N_CHUNKS
    rows_c = B // N_CHUNKS
    b2 = b.reshape(1, HIDDEN)

    table_specs = [
        pl.BlockSpec((MAX_DEPTH, HIDDEN), lambda i: (0, 0)),
        pl.BlockSpec((NUM_TYPES, HIDDEN), lambda i: (0, 0)),
        pl.BlockSpec((HIDDEN, 2 * HIDDEN), lambda i: (0, 0)),
        pl.BlockSpec((1, HIDDEN), lambda i: (0, 0)),
    ]
    comb_spec = pl.BlockSpec((1, 1, BLK_TOK), lambda i: (i, 0, 0))
    out_shape = jax.ShapeDtypeStruct((n_tok, HIDDEN), jnp.float32)

    out = None
    for ci in range(N_CHUNKS):
        d_c = depth_indices[ci * rows_c:(ci + 1) * rows_c]
        t_c = node_type_indices[ci * rows_c:(ci + 1) * rows_c]
        comb = ((t_c << 3) | d_c).reshape(grid_c, 1, BLK_TOK)
        out_spec = pl.BlockSpec((BLK_TOK, HIDDEN),
                                lambda i, _o=ci * grid_c: (i + _o, 0))
        if ci == 0:
            out = pl.pallas_call(
                _body_first,
                grid=(grid_c,),
                in_specs=[comb_spec] + table_specs,
                out_specs=out_spec,
                out_shape=out_shape,
                compiler_params=pltpu.CompilerParams(
                    dimension_semantics=("arbitrary",)),
            )(comb, depth_table, type_table, W, b2)
        else:
            out = pl.pallas_call(
                _body_chained,
                grid=(grid_c,),
                in_specs=[comb_spec] + table_specs
                + [pl.BlockSpec(memory_space=pl.ANY)],
                out_specs=out_spec,
                out_shape=out_shape,
                input_output_aliases={5: 0},
                compiler_params=pltpu.CompilerParams(
                    dimension_semantics=("arbitrary",)),
            )(comb, depth_table, type_table, W, b2, out)
    return out.reshape(B, L, HIDDEN)


# speedup vs baseline: 1.0113x; 1.0113x over previous
import jax
import jax.numpy as jnp
from jax import lax
from jax.experimental import pallas as pl
from jax.experimental.pallas import tpu as pltpu
from jax.experimental.pallas import tpu_sc as plsc

HIDDEN = 64
MAX_DEPTH = 8
NUM_TYPES = 16
K = 32

ROW_LEN = 200
LANE = 128
ROWS_PER_BLK = 128
BLK_TOK = ROWS_PER_BLK * ROW_LEN

SC_CORES = 2
SC_SUBCORES = 16
SC_WORKERS = SC_CORES * SC_SUBCORES


def _pack_body(didx_ref, tidx_ref, left_ref, right_ref):
    comb = (tidx_ref[...] << 3) | didx_ref[...]
    left_ref[...] = comb[:, :LANE]
    right_ref[:, : ROW_LEN - LANE] = comb[:, LANE:]


def _pack(depth_indices, node_type_indices):
    n_rows = depth_indices.shape[0]
    br = 2048
    half = jax.ShapeDtypeStruct((n_rows, LANE), jnp.int32)
    return pl.pallas_call(
        _pack_body,
        grid=(n_rows // br,),
        in_specs=[pl.BlockSpec((br, ROW_LEN), lambda i: (i, 0)),
                  pl.BlockSpec((br, ROW_LEN), lambda i: (i, 0))],
        out_specs=[pl.BlockSpec((br, LANE), lambda i: (i, 0)),
                   pl.BlockSpec((br, LANE), lambda i: (i, 0))],
        out_shape=[half, half],
        compiler_params=pltpu.CompilerParams(
            dimension_semantics=("arbitrary",)),
    )(depth_indices, node_type_indices)


def _sc_compact(left, right):
    n_rows = left.shape[0]
    rows_per_step = 64
    tok_per_step = rows_per_step * ROW_LEN
    steps_per_worker = n_rows // rows_per_step // SC_WORKERS
    mesh = plsc.VectorSubcoreMesh(core_axis_name="c", subcore_axis_name="s")
    vw = 16

    @pl.kernel(
        out_type=jax.ShapeDtypeStruct((n_rows * ROW_LEN,), jnp.int32),
        mesh=mesh,
        scratch_types=[
            pltpu.VMEM((rows_per_step, LANE), jnp.int32),
            pltpu.VMEM((rows_per_step, LANE), jnp.int32),
            pltpu.VMEM((tok_per_step,), jnp.int32),
            pltpu.SemaphoreType.DMA,
            pltpu.SemaphoreType.DMA,
        ],
    )
    def compact_kernel(l_hbm, r_hbm, out_hbm, lbuf, rbuf, flat, in_sem, out_sem):
        wid = lax.axis_index("s") * SC_CORES + lax.axis_index("c")

        @pl.loop(0, steps_per_worker)
        def _(h):
            g = wid * steps_per_worker + h
            r0 = g * rows_per_step
            cl = pltpu.async_copy(l_hbm.at[pl.ds(r0, rows_per_step), :],
                                  lbuf, in_sem)
            cr = pltpu.async_copy(r_hbm.at[pl.ds(r0, rows_per_step), :],
                                  rbuf, in_sem)
            cl.wait()
            cr.wait()
            for r in range(rows_per_step):
                base = r * ROW_LEN
                for j in range(LANE // vw):
                    flat[pl.ds(base + j * vw, vw)] = lbuf[r, pl.ds(j * vw, vw)]
                for j in range((ROW_LEN - LANE) // vw):
                    flat[pl.ds(base + LANE + j * vw, vw)] = rbuf[r, pl.ds(j * vw, vw)]
                tail = ROW_LEN - LANE - (ROW_LEN - LANE) // vw * vw
                if tail:
                    c0 = ROW_LEN - LANE - vw
                    flat[pl.ds(base + LANE + c0, vw)] = rbuf[r, pl.ds(c0, vw)]
            pltpu.async_copy(
                flat, out_hbm.at[pl.ds(r0 * ROW_LEN, tok_per_step)], out_sem
            ).wait()

    return compact_kernel(left, right)


def _main_body(comb_ref, dtab_ref, ttab_ref, w_ref, b_ref, out_ref):
    w = w_ref[...]
    pd = lax.dot_general(dtab_ref[...], w[:, :HIDDEN],
                         (((1,), (1,)), ((), ())),
                         preferred_element_type=jnp.float32) + b_ref[...]
    pt = lax.dot_general(ttab_ref[...], w[:, HIDDEN:],
                         (((1,), (1,)), ((), ())),
                         preferred_element_type=jnp.float32)
    ptab = jnp.concatenate(
        [pd, pt, jnp.zeros((K - MAX_DEPTH - NUM_TYPES, HIDDEN), jnp.float32)], axis=0)

    row = comb_ref[...][None, :]
    d = jnp.broadcast_to(row & (MAX_DEPTH - 1), (K, BLK_TOK))
    c = jnp.broadcast_to((row >> 3) + MAX_DEPTH, (K, BLK_TOK))
    iota = lax.broadcasted_iota(jnp.int32, (K, BLK_TOK), 0)
    two_hot_t = jnp.where((iota == d) | (iota == c), 1.0, 0.0)
    out_ref[...] = lax.dot_general(two_hot_t, ptab,
                                   (((0,), (0,)), ((), ())),
                                   preferred_element_type=jnp.float32)


def kernel(depth_indices, node_type_indices, depth_table, type_table, W, b):
    B, L = depth_indices.shape
    n_tok = B * L
    grid = n_tok // BLK_TOK

    left, right = _pack(depth_indices, node_type_indices)
    comb_flat = _sc_compact(left, right)

    out = pl.pallas_call(
        _main_body,
        grid=(grid,),
        in_specs=[
            pl.BlockSpec((BLK_TOK,), lambda i: (i,)),
            pl.BlockSpec((MAX_DEPTH, HIDDEN), lambda i: (0, 0)),
            pl.BlockSpec((NUM_TYPES, HIDDEN), lambda i: (0, 0)),
            pl.BlockSpec((HIDDEN, 2 * HIDDEN), lambda i: (0, 0)),
            pl.BlockSpec((1, HIDDEN), lambda i: (0, 0)),
        ],
        out_specs=pl.BlockSpec((BLK_TOK, HIDDEN), lambda i: (i, 0)),
        out_shape=jax.ShapeDtypeStruct((n_tok, HIDDEN), jnp.float32),
        compiler_params=pltpu.CompilerParams(
            dimension_semantics=("arbitrary",)),
    )(comb_flat, depth_table, type_table, W, b.reshape(1, HIDDEN))
    return out.reshape(B, L, HIDDEN)


# speedup vs baseline: 1.0128x; 1.0015x over previous
import jax
import jax.numpy as jnp
from jax import lax
from jax.experimental import pallas as pl
from jax.experimental.pallas import tpu as pltpu
from jax.experimental.pallas import tpu_sc as plsc

HIDDEN = 64
MAX_DEPTH = 8
NUM_TYPES = 16
K = 32

ROW_LEN = 200
LANE = 128
ROWS_PER_BLK = 128
BLK_TOK = ROWS_PER_BLK * ROW_LEN

SC_CORES = 2
SC_SUBCORES = 16
SC_WORKERS = SC_CORES * SC_SUBCORES


def _pack_body(didx_ref, tidx_ref, left_ref, right_ref):
    comb = (tidx_ref[...] << 3) | didx_ref[...]
    left_ref[...] = comb[:, :LANE]
    right_ref[:, : ROW_LEN - LANE] = comb[:, LANE:]


def _pack(depth_indices, node_type_indices):
    n_rows = depth_indices.shape[0]
    br = 2048
    half = jax.ShapeDtypeStruct((n_rows, LANE), jnp.int32)
    return pl.pallas_call(
        _pack_body,
        grid=(n_rows // br,),
        in_specs=[pl.BlockSpec((br, ROW_LEN), lambda i: (i, 0)),
                  pl.BlockSpec((br, ROW_LEN), lambda i: (i, 0))],
        out_specs=[pl.BlockSpec((br, LANE), lambda i: (i, 0)),
                   pl.BlockSpec((br, LANE), lambda i: (i, 0))],
        out_shape=[half, half],
        compiler_params=pltpu.CompilerParams(
            dimension_semantics=("arbitrary",)),
    )(depth_indices, node_type_indices)


def _sc_compact(left, right):
    n_rows = left.shape[0] // LANE
    rows_per_step = 64
    tok_per_step = rows_per_step * ROW_LEN
    steps_per_worker = n_rows // rows_per_step // SC_WORKERS
    mesh = plsc.VectorSubcoreMesh(core_axis_name="c", subcore_axis_name="s")
    vw = 16

    @pl.kernel(
        out_type=jax.ShapeDtypeStruct((n_rows * ROW_LEN,), jnp.int32),
        mesh=mesh,
        scratch_types=[
            pltpu.VMEM((rows_per_step * LANE,), jnp.int32),
            pltpu.VMEM((rows_per_step * LANE,), jnp.int32),
            pltpu.VMEM((tok_per_step,), jnp.int32),
            pltpu.SemaphoreType.DMA,
            pltpu.SemaphoreType.DMA,
        ],
    )
    def compact_kernel(l_hbm, r_hbm, out_hbm, lbuf, rbuf, flat, in_sem, out_sem):
        wid = lax.axis_index("s") * SC_CORES + lax.axis_index("c")

        @pl.loop(0, steps_per_worker)
        def _(h):
            g = wid * steps_per_worker + h
            r0 = g * rows_per_step
            cl = pltpu.async_copy(l_hbm.at[pl.ds(r0 * LANE, rows_per_step * LANE)],
                                  lbuf, in_sem)
            cr = pltpu.async_copy(r_hbm.at[pl.ds(r0 * LANE, rows_per_step * LANE)],
                                  rbuf, in_sem)
            cl.wait()
            cr.wait()
            for r in range(rows_per_step):
                base = r * ROW_LEN
                src = r * LANE
                for j in range(LANE // vw):
                    flat[pl.ds(base + j * vw, vw)] = lbuf[pl.ds(src + j * vw, vw)]
                for j in range((ROW_LEN - LANE) // vw):
                    flat[pl.ds(base + LANE + j * vw, vw)] = rbuf[pl.ds(src + j * vw, vw)]
                tail = ROW_LEN - LANE - (ROW_LEN - LANE) // vw * vw
                if tail:
                    c0 = ROW_LEN - LANE - vw
                    flat[pl.ds(base + LANE + c0, vw)] = rbuf[pl.ds(src + c0, vw)]
            pltpu.async_copy(
                flat, out_hbm.at[pl.ds(r0 * ROW_LEN, tok_per_step)], out_sem
            ).wait()

    return compact_kernel(left, right)


def _main_body(comb_ref, dtab_ref, ttab_ref, w_ref, b_ref, out_ref):
    w = w_ref[...]
    pd = lax.dot_general(dtab_ref[...], w[:, :HIDDEN],
                         (((1,), (1,)), ((), ())),
                         preferred_element_type=jnp.float32) + b_ref[...]
    pt = lax.dot_general(ttab_ref[...], w[:, HIDDEN:],
                         (((1,), (1,)), ((), ())),
                         preferred_element_type=jnp.float32)
    ptab = jnp.concatenate(
        [pd, pt, jnp.zeros((K - MAX_DEPTH - NUM_TYPES, HIDDEN), jnp.float32)], axis=0)

    row = comb_ref[...][None, :]
    d = jnp.broadcast_to(row & (MAX_DEPTH - 1), (K, BLK_TOK))
    c = jnp.broadcast_to((row >> 3) + MAX_DEPTH, (K, BLK_TOK))
    iota = lax.broadcasted_iota(jnp.int32, (K, BLK_TOK), 0)
    two_hot_t = jnp.where((iota == d) | (iota == c), 1.0, 0.0)
    out_ref[...] = lax.dot_general(two_hot_t, ptab,
                                   (((0,), (0,)), ((), ())),
                                   preferred_element_type=jnp.float32)


def kernel(depth_indices, node_type_indices, depth_table, type_table, W, b):
    B, L = depth_indices.shape
    n_tok = B * L
    grid = n_tok // BLK_TOK

    left, right = _pack(depth_indices, node_type_indices)
    comb_flat = _sc_compact(left.reshape(-1), right.reshape(-1))

    out = pl.pallas_call(
        _main_body,
        grid=(grid,),
        in_specs=[
            pl.BlockSpec((BLK_TOK,), lambda i: (i,)),
            pl.BlockSpec((MAX_DEPTH, HIDDEN), lambda i: (0, 0)),
            pl.BlockSpec((NUM_TYPES, HIDDEN), lambda i: (0, 0)),
            pl.BlockSpec((HIDDEN, 2 * HIDDEN), lambda i: (0, 0)),
            pl.BlockSpec((1, HIDDEN), lambda i: (0, 0)),
        ],
        out_specs=pl.BlockSpec((BLK_TOK, HIDDEN), lambda i: (i, 0)),
        out_shape=jax.ShapeDtypeStruct((n_tok, HIDDEN), jnp.float32),
        compiler_params=pltpu.CompilerParams(
            dimension_semantics=("arbitrary",)),
    )(comb_flat, depth_table, type_table, W, b.reshape(1, HIDDEN))
    return out.reshape(B, L, HIDDEN)
